# Initial kernel scaffold; baseline (speedup 1.0000x reference)
import functools

import jax
import jax.numpy as jnp
from jax import lax
from jax.experimental import pallas as pl
from jax.experimental.pallas import tpu as pltpu
from jax.experimental.pallas import tpu_sc as plsc

NU = 25000
NPAD = 25088
ROWS = 2 * NPAD
TROWS = 2 * ROWS
HALF = 32
DIM = 64
E = 800_000
EP = 819_200
EPR = EP // 128
TILE_ROWS = EPR // 16
NBLK = TILE_ROWS // 8
ACC_SLICE = ROWS // 16
NBATCH = 1024
UPT = NBATCH // 16

_mesh = plsc.VectorSubcoreMesh(
    core_axis_name="c", subcore_axis_name="s", num_cores=2, num_subcores=16)


def _propagate_body(tin, src2, dst2, w2, zrows, tout,
                    sbufs, dbufs, wbufs, rows, acc, gsem, ssem):
    c = lax.axis_index("c")
    s = lax.axis_index("s")
    coff = c * ROWS
    accbase = s * ACC_SLICE
    pltpu.sync_copy(zrows.at[pl.ds(accbase, ACC_SLICE)],
                    acc.at[pl.ds(accbase, ACC_SLICE)])
    plsc.subcore_barrier()

    coffv = jnp.full((16,), coff, jnp.int32)
    k_nu = jnp.full((16,), NU, jnp.int32)
    k_pad = jnp.full((16,), NPAD - NU, jnp.int32)
    k_zero = jnp.zeros((16,), jnp.int32)
    iota = lax.iota(jnp.int32, 16)
    rowbase = s * TILE_ROWS

    @pl.loop(0, NBLK)
    def _block(b):
        r0 = rowbase + b * 8
        pltpu.sync_copy(src2.at[pl.ds(r0, 8)], sbufs)
        pltpu.sync_copy(dst2.at[pl.ds(r0, 8)], dbufs)
        pltpu.sync_copy(w2.at[pl.ds(r0, 8)], wbufs)

        for k in range(8):
            for g in range(8):
                sv = sbufs[k, pl.ds(g * 16, 16)]
                sv = sv + jnp.where(sv >= k_nu, k_pad, k_zero) + coffv
                sbufs[k, pl.ds(g * 16, 16)] = sv
                dv = dbufs[k, pl.ds(g * 16, 16)]
                dv = dv + jnp.where(dv >= k_nu, k_pad, k_zero)
                dbufs[k, pl.ds(g * 16, 16)] = dv

        gcps = [pltpu.async_copy(tin.at[sbufs.at[k]],
                                 rows.at[pl.ds(k * 128, 128)], gsem)
                for k in range(8)]
        for cp in gcps:
            cp.wait()

        @pl.loop(0, 8)
        def _scale(k):
            kv = jnp.full((16,), k, jnp.int32)
            for g in range(8):
                gid = g * 16 + iota
                wv = plsc.load_gather(wbufs, [kv, gid])
                rid = k * 128 + gid
                for d in range(HALF):
                    cid = jnp.full((16,), d, jnp.int32)
                    col = plsc.load_gather(rows, [rid, cid])
                    plsc.store_scatter(rows, [rid, cid], col * wv)

        scps = [pltpu.async_copy(rows.at[pl.ds(k * 128, 128)],
                                 acc.at[dbufs.at[k]], ssem, add=True)
                for k in range(8)]
        for cp in scps:
            cp.wait()

    plsc.subcore_barrier()
    pltpu.sync_copy(acc.at[pl.ds(accbase, ACC_SLICE)],
                    tout.at[pl.ds(coff + accbase, ACC_SLICE)])


_propagate = functools.partial(
    pl.kernel,
    out_type=jax.ShapeDtypeStruct((TROWS, HALF), jnp.float32),
    mesh=_mesh,
    scratch_types=[
        pltpu.VMEM((8, 128), jnp.int32),
        pltpu.VMEM((8, 128), jnp.int32),
        pltpu.VMEM((8, 128), jnp.float32),
        pltpu.VMEM((1024, HALF), jnp.float32),
        pltpu.VMEM_SHARED((ROWS, HALF), jnp.float32),
        pltpu.SemaphoreType.DMA,
        pltpu.SemaphoreType.DMA,
    ],
)(_propagate_body)


def _users_body(t0, t1, t2, t3, users_h, uout, ubuf, gacc, gtmp, usem):
    c = lax.axis_index("c")
    s = lax.axis_index("s")
    coffv = jnp.full((16,), c * ROWS, jnp.int32)
    ubase = s * UPT
    pltpu.sync_copy(users_h.at[pl.ds(ubase, UPT)], ubuf)
    for g in range(UPT // 16):
        uv = ubuf[pl.ds(g * 16, 16)] + coffv
        ubuf[pl.ds(g * 16, 16)] = uv
    pltpu.async_copy(t0.at[ubuf], gacc, usem).wait()
    for tk in (t1, t2, t3):
        pltpu.async_copy(tk.at[ubuf], gtmp, usem).wait()
        for r in range(UPT):
            for h in range(HALF // 16):
                gacc[r, pl.ds(h * 16, 16)] = (
                    gacc[r, pl.ds(h * 16, 16)] + gtmp[r, pl.ds(h * 16, 16)])
    quarter = jnp.full((16,), 0.25, jnp.float32)
    for r in range(UPT):
        for h in range(HALF // 16):
            gacc[r, pl.ds(h * 16, 16)] = gacc[r, pl.ds(h * 16, 16)] * quarter
    pltpu.sync_copy(gacc, uout.at[pl.ds(c * NBATCH + ubase, UPT)])


_users_gather = functools.partial(
    pl.kernel,
    out_type=jax.ShapeDtypeStruct((2 * NBATCH, HALF), jnp.float32),
    mesh=_mesh,
    scratch_types=[
        pltpu.VMEM((UPT,), jnp.int32),
        pltpu.VMEM((UPT, HALF), jnp.float32),
        pltpu.VMEM((UPT, HALF), jnp.float32),
        pltpu.SemaphoreType.DMA,
    ],
)(_users_body)


def _rating_body(t0b, t1b, t2b, t3b, ub, ob):
    it = (t0b[...] + t1b[...] + t2b[...] + t3b[...]) * 0.25
    u = ub[...]
    dn = (((1,), (1,)), ((), ()))
    r = (lax.dot_general(u[0], it[0], dn, preferred_element_type=jnp.float32)
         + lax.dot_general(u[1], it[1], dn, preferred_element_type=jnp.float32))
    ob[...] = 1.0 / (1.0 + jnp.exp(-r))


def _rating(t0, t1, t2, t3, u):
    tspec = pl.BlockSpec((2, 512, HALF), lambda n: (0, NPAD // 512 + n, 0))
    return pl.pallas_call(
        _rating_body,
        grid=(NPAD // 512,),
        in_specs=[tspec, tspec, tspec, tspec,
                  pl.BlockSpec((2, NBATCH, HALF), lambda n: (0, 0, 0))],
        out_specs=pl.BlockSpec((NBATCH, 512), lambda n: (0, n)),
        out_shape=jax.ShapeDtypeStruct((NBATCH, NU), jnp.float32),
    )(t0, t1, t2, t3, u)


def kernel(user_emb, item_emb, edge_weight, edge_index, users):
    src = edge_index[0].astype(jnp.int32)
    dst = edge_index[1].astype(jnp.int32)
    pad = EP - E
    src2 = jnp.concatenate([src, jnp.zeros((pad,), jnp.int32)]).reshape(EPR, 128)
    dst2 = jnp.concatenate([dst, jnp.zeros((pad,), jnp.int32)]).reshape(EPR, 128)
    w2 = jnp.concatenate(
        [edge_weight, jnp.zeros((pad,), jnp.float32)]).reshape(EPR, 128)

    t0 = jnp.zeros((2, ROWS, HALF), jnp.float32)
    t0 = t0.at[0, :NU].set(user_emb[:, :HALF])
    t0 = t0.at[0, NPAD:NPAD + NU].set(item_emb[:, :HALF])
    t0 = t0.at[1, :NU].set(user_emb[:, HALF:])
    t0 = t0.at[1, NPAD:NPAD + NU].set(item_emb[:, HALF:])
    t0 = t0.reshape(TROWS, HALF)
    zrows = jnp.zeros((ROWS, HALF), jnp.float32)

    t1 = _propagate(t0, src2, dst2, w2, zrows)
    t2 = _propagate(t1, src2, dst2, w2, zrows)
    t3 = _propagate(t2, src2, dst2, w2, zrows)
    u = _users_gather(t0, t1, t2, t3, users.astype(jnp.int32))

    return _rating(t0.reshape(2, ROWS, HALF), t1.reshape(2, ROWS, HALF),
                   t2.reshape(2, ROWS, HALF), t3.reshape(2, ROWS, HALF),
                   u.reshape(2, NBATCH, HALF))


# speedup vs baseline: 5.6631x; 5.6631x over previous
import functools

import jax
import jax.numpy as jnp
from jax import lax
from jax.experimental import pallas as pl
from jax.experimental.pallas import tpu as pltpu
from jax.experimental.pallas import tpu_sc as plsc

NU = 25000
NPAD = 25088
ROWS = 2 * NPAD
TROWS = 2 * ROWS
HALF = 32
DIM = 64
E = 800_000
EP = 819_200
EPR = EP // 128
TILE_ROWS = EPR // 16
BLKROWS = 2
NBLK = TILE_ROWS // BLKROWS
ACC_SLICE = ROWS // 16
NBATCH = 1024
UPT = NBATCH // 16

_mesh = plsc.VectorSubcoreMesh(
    core_axis_name="c", subcore_axis_name="s", num_cores=2, num_subcores=16)


def _propagate_body(tin, ip3, zrows, tout,
                    ia, ib, rowsa, rowsb, rows2, acc, gsa, gsb, ssem):
    c = lax.axis_index("c")
    s = lax.axis_index("s")
    coff = c * ROWS
    accbase = s * ACC_SLICE
    pltpu.sync_copy(zrows.at[pl.ds(accbase, ACC_SLICE)],
                    acc.at[pl.ds(accbase, ACC_SLICE)])
    plsc.subcore_barrier()

    coffv = jnp.full((16,), coff, jnp.int32)
    k_nu = jnp.full((16,), NU, jnp.int32)
    k_pad = jnp.full((16,), NPAD - NU, jnp.int32)
    k_zero = jnp.zeros((16,), jnp.int32)
    iota = lax.iota(jnp.int32, 16)
    rowbase = s * TILE_ROWS

    def load_block(ix, r0):
        r0c = jnp.minimum(r0, EPR - BLKROWS)
        pltpu.sync_copy(ip3.at[pl.ds(r0c * 3, 3 * BLKROWS)], ix)
        for k in range(BLKROWS):
            for g in range(8):
                sv = ix[3 * k, pl.ds(g * 16, 16)]
                sv = sv + jnp.where(sv >= k_nu, k_pad, k_zero) + coffv
                ix[3 * k, pl.ds(g * 16, 16)] = sv
                dv = ix[3 * k + 1, pl.ds(g * 16, 16)]
                ix[3 * k + 1, pl.ds(g * 16, 16)] = (
                    dv + jnp.where(dv >= k_nu, k_pad, k_zero))

    def fire_gather(ix, rowsx, sem):
        for k in range(BLKROWS):
            pltpu.async_copy(tin.at[ix.at[3 * k]],
                             rowsx.at[pl.ds(k * 128, 128)], sem)

    def drain_gather(ix, rowsx, sem):
        for k in range(BLKROWS):
            pltpu.make_async_copy(tin.at[ix.at[3 * k]],
                                  rowsx.at[pl.ds(k * 128, 128)], sem).wait()

    def scale_and_scatter(ix, rowsx):
        for k in range(BLKROWS):

            @pl.loop(0, 8)
            def _sg(g):
                wvec = plsc.bitcast(ix[3 * k + 2, pl.ds(g * 16, 16)],
                                    jnp.float32)
                base = k * 128 + g * 16
                for e in range(16):
                    ev = jnp.full((16,), e, jnp.int32)
                    wv = jnp.take_along_axis(
                        wvec, ev, axis=0, mode="promise_in_bounds")
                    r = base + e
                    va = rowsx[r, pl.ds(0, 16)] * wv
                    vb = rowsx[r, pl.ds(16, 16)] * wv
                    rows2[r, pl.ds(0, 16)] = va
                    rows2[r, pl.ds(16, 16)] = vb
        scps = [pltpu.async_copy(rows2.at[pl.ds(k * 128, 128)],
                                 acc.at[ix.at[3 * k + 1]], ssem, add=True)
                for k in range(BLKROWS)]
        for cp in scps:
            cp.wait()

    load_block(ia, rowbase)
    fire_gather(ia, rowsa, gsa)

    @pl.loop(0, NBLK // 2)
    def _iter(i):
        r0 = rowbase + i * 2 * BLKROWS
        load_block(ib, r0 + BLKROWS)
        fire_gather(ib, rowsb, gsb)
        drain_gather(ia, rowsa, gsa)
        scale_and_scatter(ia, rowsa)
        load_block(ia, r0 + 2 * BLKROWS)
        fire_gather(ia, rowsa, gsa)
        drain_gather(ib, rowsb, gsb)
        scale_and_scatter(ib, rowsb)

    drain_gather(ia, rowsa, gsa)

    plsc.subcore_barrier()
    pltpu.sync_copy(acc.at[pl.ds(accbase, ACC_SLICE)],
                    tout.at[pl.ds(coff + accbase, ACC_SLICE)])


_propagate = functools.partial(
    pl.kernel,
    out_type=jax.ShapeDtypeStruct((TROWS, HALF), jnp.float32),
    mesh=_mesh,
    compiler_params=pltpu.CompilerParams(needs_layout_passes=False, use_tc_tiling_on_sc=False),
    scratch_types=[
        pltpu.VMEM((3 * BLKROWS, 128), jnp.int32),
        pltpu.VMEM((3 * BLKROWS, 128), jnp.int32),
        pltpu.VMEM((BLKROWS * 128, HALF), jnp.float32),
        pltpu.VMEM((BLKROWS * 128, HALF), jnp.float32),
        pltpu.VMEM((BLKROWS * 128, HALF), jnp.float32),
        pltpu.VMEM_SHARED((ROWS, HALF), jnp.float32),
        pltpu.SemaphoreType.DMA,
        pltpu.SemaphoreType.DMA,
        pltpu.SemaphoreType.DMA,
    ],
)(_propagate_body)


def _users_body(t0, t1, t2, t3, users_h, uout, ubuf, gacc, gtmp, usem):
    c = lax.axis_index("c")
    s = lax.axis_index("s")
    coffv = jnp.full((16,), c * ROWS, jnp.int32)
    ubase = s * UPT
    pltpu.sync_copy(users_h.at[pl.ds(ubase, UPT)], ubuf)
    for g in range(UPT // 16):
        uv = ubuf[pl.ds(g * 16, 16)] + coffv
        ubuf[pl.ds(g * 16, 16)] = uv
    pltpu.async_copy(t0.at[ubuf], gacc, usem).wait()
    for tk in (t1, t2, t3):
        pltpu.async_copy(tk.at[ubuf], gtmp, usem).wait()
        for r in range(UPT):
            for h in range(HALF // 16):
                gacc[r, pl.ds(h * 16, 16)] = (
                    gacc[r, pl.ds(h * 16, 16)] + gtmp[r, pl.ds(h * 16, 16)])
    quarter = jnp.full((16,), 0.25, jnp.float32)
    for r in range(UPT):
        for h in range(HALF // 16):
            gacc[r, pl.ds(h * 16, 16)] = gacc[r, pl.ds(h * 16, 16)] * quarter
    pltpu.sync_copy(gacc, uout.at[pl.ds(c * NBATCH + ubase, UPT)])


_users_gather = functools.partial(
    pl.kernel,
    out_type=jax.ShapeDtypeStruct((2 * NBATCH, HALF), jnp.float32),
    mesh=_mesh,
    compiler_params=pltpu.CompilerParams(needs_layout_passes=False, use_tc_tiling_on_sc=False),
    scratch_types=[
        pltpu.VMEM((UPT,), jnp.int32),
        pltpu.VMEM((UPT, HALF), jnp.float32),
        pltpu.VMEM((UPT, HALF), jnp.float32),
        pltpu.SemaphoreType.DMA,
    ],
)(_users_body)


def _rating_body(t0b, t1b, t2b, t3b, ub, ob):
    it = (t0b[...] + t1b[...] + t2b[...] + t3b[...]) * 0.25
    u = ub[...]
    dn = (((1,), (1,)), ((), ()))
    r = (lax.dot_general(u[0], it[0], dn, preferred_element_type=jnp.float32)
         + lax.dot_general(u[1], it[1], dn, preferred_element_type=jnp.float32))
    ob[...] = 1.0 / (1.0 + jnp.exp(-r))


def _rating(t0, t1, t2, t3, u):
    tspec = pl.BlockSpec((2, 512, HALF), lambda n: (0, NPAD // 512 + n, 0))
    return pl.pallas_call(
        _rating_body,
        grid=(NPAD // 512,),
        in_specs=[tspec, tspec, tspec, tspec,
                  pl.BlockSpec((2, NBATCH, HALF), lambda n: (0, 0, 0))],
        out_specs=pl.BlockSpec((NBATCH, 512), lambda n: (0, n)),
        out_shape=jax.ShapeDtypeStruct((NBATCH, NU), jnp.float32),
    )(t0, t1, t2, t3, u)


def kernel(user_emb, item_emb, edge_weight, edge_index, users):
    src = edge_index[0].astype(jnp.int32)
    dst = edge_index[1].astype(jnp.int32)
    pad = EP - E
    src2 = jnp.concatenate([src, jnp.zeros((pad,), jnp.int32)]).reshape(EPR, 128)
    dst2 = jnp.concatenate([dst, jnp.zeros((pad,), jnp.int32)]).reshape(EPR, 128)
    w2 = jnp.concatenate(
        [edge_weight, jnp.zeros((pad,), jnp.float32)]).reshape(EPR, 128)
    wbits = lax.bitcast_convert_type(w2, jnp.int32)
    ip3 = jnp.stack([src2, dst2, wbits], axis=1).reshape(EPR * 3, 128)

    t0 = jnp.zeros((2, ROWS, HALF), jnp.float32)
    t0 = t0.at[0, :NU].set(user_emb[:, :HALF])
    t0 = t0.at[0, NPAD:NPAD + NU].set(item_emb[:, :HALF])
    t0 = t0.at[1, :NU].set(user_emb[:, HALF:])
    t0 = t0.at[1, NPAD:NPAD + NU].set(item_emb[:, HALF:])
    t0 = t0.reshape(TROWS, HALF)
    zrows = jnp.zeros((ROWS, HALF), jnp.float32)

    t1 = _propagate(t0, ip3, zrows)
    t2 = _propagate(t1, ip3, zrows)
    t3 = _propagate(t2, ip3, zrows)
    u = _users_gather(t0, t1, t2, t3, users.astype(jnp.int32))

    return _rating(t0.reshape(2, ROWS, HALF), t1.reshape(2, ROWS, HALF),
                   t2.reshape(2, ROWS, HALF), t3.reshape(2, ROWS, HALF),
                   u.reshape(2, NBATCH, HALF))


# speedup vs baseline: 5.7376x; 1.0132x over previous
import functools

import jax
import jax.numpy as jnp
from jax import lax
from jax.experimental import pallas as pl
from jax.experimental.pallas import tpu as pltpu
from jax.experimental.pallas import tpu_sc as plsc

NU = 25000
NPAD = 25088
ROWS = 2 * NPAD
TROWS = 2 * ROWS
HALF = 32
DIM = 64
E = 800_000
EP = 823_296
EPR = EP // 128
TILE_ROWS = EPR // 16
BLKROWS = 3
NBLK = TILE_ROWS // BLKROWS
ACC_SLICE = ROWS // 16
NBATCH = 1024
UPT = NBATCH // 16

_mesh = plsc.VectorSubcoreMesh(
    core_axis_name="c", subcore_axis_name="s", num_cores=2, num_subcores=16)


def _propagate_body(tin, ip3, zrows, tout,
                    ia, ib, rowsa, rowsb, acc, gsa, gsb, ssem):
    c = lax.axis_index("c")
    s = lax.axis_index("s")
    coff = c * ROWS
    accbase = s * ACC_SLICE
    pltpu.sync_copy(zrows.at[pl.ds(accbase, ACC_SLICE)],
                    acc.at[pl.ds(accbase, ACC_SLICE)])
    plsc.subcore_barrier()

    coffv = jnp.full((16,), coff, jnp.int32)
    k_nu = jnp.full((16,), NU, jnp.int32)
    k_pad = jnp.full((16,), NPAD - NU, jnp.int32)
    k_zero = jnp.zeros((16,), jnp.int32)
    iota = lax.iota(jnp.int32, 16)
    rowbase = s * TILE_ROWS

    def load_block(ix, r0):
        r0c = jnp.minimum(r0, EPR - BLKROWS)
        pltpu.sync_copy(ip3.at[pl.ds(r0c * 3, 3 * BLKROWS)], ix)
        for k in range(BLKROWS):
            for g in range(8):
                sv = ix[3 * k, pl.ds(g * 16, 16)]
                sv = sv + jnp.where(sv >= k_nu, k_pad, k_zero) + coffv
                ix[3 * k, pl.ds(g * 16, 16)] = sv
                dv = ix[3 * k + 1, pl.ds(g * 16, 16)]
                ix[3 * k + 1, pl.ds(g * 16, 16)] = (
                    dv + jnp.where(dv >= k_nu, k_pad, k_zero))

    def fire_gather(ix, rowsx, sem):
        for k in range(BLKROWS):
            pltpu.async_copy(tin.at[ix.at[3 * k]],
                             rowsx.at[pl.ds(k * 128, 128)], sem)

    def drain_gather(ix, rowsx, sem):
        for k in range(BLKROWS):
            pltpu.make_async_copy(tin.at[ix.at[3 * k]],
                                  rowsx.at[pl.ds(k * 128, 128)], sem).wait()

    def scale_and_scatter(ix, rowsx):
        for k in range(BLKROWS):

            @pl.loop(0, 8)
            def _sg(g):
                wvec = plsc.bitcast(ix[3 * k + 2, pl.ds(g * 16, 16)],
                                    jnp.float32)
                base = k * 128 + g * 16
                for e in range(16):
                    ev = jnp.full((16,), e, jnp.int32)
                    wv = jnp.take_along_axis(
                        wvec, ev, axis=0, mode="promise_in_bounds")
                    r = base + e
                    rowsx[r, pl.ds(0, 16)] = rowsx[r, pl.ds(0, 16)] * wv
                    rowsx[r, pl.ds(16, 16)] = rowsx[r, pl.ds(16, 16)] * wv
        scps = [pltpu.async_copy(rowsx.at[pl.ds(k * 128, 128)],
                                 acc.at[ix.at[3 * k + 1]], ssem, add=True)
                for k in range(BLKROWS)]
        for cp in scps:
            cp.wait()

    load_block(ia, rowbase)
    fire_gather(ia, rowsa, gsa)

    @pl.loop(0, NBLK // 2)
    def _iter(i):
        r0 = rowbase + i * 2 * BLKROWS
        load_block(ib, r0 + BLKROWS)
        fire_gather(ib, rowsb, gsb)
        drain_gather(ia, rowsa, gsa)
        scale_and_scatter(ia, rowsa)
        load_block(ia, r0 + 2 * BLKROWS)
        fire_gather(ia, rowsa, gsa)
        drain_gather(ib, rowsb, gsb)
        scale_and_scatter(ib, rowsb)

    drain_gather(ia, rowsa, gsa)

    plsc.subcore_barrier()
    pltpu.sync_copy(acc.at[pl.ds(accbase, ACC_SLICE)],
                    tout.at[pl.ds(coff + accbase, ACC_SLICE)])


_propagate = functools.partial(
    pl.kernel,
    out_type=jax.ShapeDtypeStruct((TROWS, HALF), jnp.float32),
    mesh=_mesh,
    compiler_params=pltpu.CompilerParams(needs_layout_passes=False, use_tc_tiling_on_sc=False),
    scratch_types=[
        pltpu.VMEM((3 * BLKROWS, 128), jnp.int32),
        pltpu.VMEM((3 * BLKROWS, 128), jnp.int32),
        pltpu.VMEM((BLKROWS * 128, HALF), jnp.float32),
        pltpu.VMEM((BLKROWS * 128, HALF), jnp.float32),
        pltpu.VMEM_SHARED((ROWS, HALF), jnp.float32),
        pltpu.SemaphoreType.DMA,
        pltpu.SemaphoreType.DMA,
        pltpu.SemaphoreType.DMA,
    ],
)(_propagate_body)


def _users_body(t0, t1, t2, t3, users_h, uout, ubuf, gacc, gtmp, usem):
    c = lax.axis_index("c")
    s = lax.axis_index("s")
    coffv = jnp.full((16,), c * ROWS, jnp.int32)
    ubase = s * UPT
    pltpu.sync_copy(users_h.at[pl.ds(ubase, UPT)], ubuf)
    for g in range(UPT // 16):
        uv = ubuf[pl.ds(g * 16, 16)] + coffv
        ubuf[pl.ds(g * 16, 16)] = uv
    pltpu.async_copy(t0.at[ubuf], gacc, usem).wait()
    for tk in (t1, t2, t3):
        pltpu.async_copy(tk.at[ubuf], gtmp, usem).wait()
        for r in range(UPT):
            for h in range(HALF // 16):
                gacc[r, pl.ds(h * 16, 16)] = (
                    gacc[r, pl.ds(h * 16, 16)] + gtmp[r, pl.ds(h * 16, 16)])
    quarter = jnp.full((16,), 0.25, jnp.float32)
    for r in range(UPT):
        for h in range(HALF // 16):
            gacc[r, pl.ds(h * 16, 16)] = gacc[r, pl.ds(h * 16, 16)] * quarter
    pltpu.sync_copy(gacc, uout.at[pl.ds(c * NBATCH + ubase, UPT)])


_users_gather = functools.partial(
    pl.kernel,
    out_type=jax.ShapeDtypeStruct((2 * NBATCH, HALF), jnp.float32),
    mesh=_mesh,
    compiler_params=pltpu.CompilerParams(needs_layout_passes=False, use_tc_tiling_on_sc=False),
    scratch_types=[
        pltpu.VMEM((UPT,), jnp.int32),
        pltpu.VMEM((UPT, HALF), jnp.float32),
        pltpu.VMEM((UPT, HALF), jnp.float32),
        pltpu.SemaphoreType.DMA,
    ],
)(_users_body)


def _rating_body(t0b, t1b, t2b, t3b, ub, ob):
    it = (t0b[...] + t1b[...] + t2b[...] + t3b[...]) * 0.25
    u = ub[...]
    dn = (((1,), (1,)), ((), ()))
    r = (lax.dot_general(u[0], it[0], dn, preferred_element_type=jnp.float32)
         + lax.dot_general(u[1], it[1], dn, preferred_element_type=jnp.float32))
    ob[...] = 1.0 / (1.0 + jnp.exp(-r))


def _rating(t0, t1, t2, t3, u):
    tspec = pl.BlockSpec((2, 512, HALF), lambda n: (0, NPAD // 512 + n, 0))
    return pl.pallas_call(
        _rating_body,
        grid=(NPAD // 512,),
        in_specs=[tspec, tspec, tspec, tspec,
                  pl.BlockSpec((2, NBATCH, HALF), lambda n: (0, 0, 0))],
        out_specs=pl.BlockSpec((NBATCH, 512), lambda n: (0, n)),
        out_shape=jax.ShapeDtypeStruct((NBATCH, NU), jnp.float32),
    )(t0, t1, t2, t3, u)


def kernel(user_emb, item_emb, edge_weight, edge_index, users):
    src = edge_index[0].astype(jnp.int32)
    dst = edge_index[1].astype(jnp.int32)
    pad = EP - E
    src2 = jnp.concatenate([src, jnp.zeros((pad,), jnp.int32)]).reshape(EPR, 128)
    dst2 = jnp.concatenate([dst, jnp.zeros((pad,), jnp.int32)]).reshape(EPR, 128)
    w2 = jnp.concatenate(
        [edge_weight, jnp.zeros((pad,), jnp.float32)]).reshape(EPR, 128)
    wbits = lax.bitcast_convert_type(w2, jnp.int32)
    ip3 = jnp.stack([src2, dst2, wbits], axis=1).reshape(EPR * 3, 128)

    t0 = jnp.zeros((2, ROWS, HALF), jnp.float32)
    t0 = t0.at[0, :NU].set(user_emb[:, :HALF])
    t0 = t0.at[0, NPAD:NPAD + NU].set(item_emb[:, :HALF])
    t0 = t0.at[1, :NU].set(user_emb[:, HALF:])
    t0 = t0.at[1, NPAD:NPAD + NU].set(item_emb[:, HALF:])
    t0 = t0.reshape(TROWS, HALF)
    zrows = jnp.zeros((ROWS, HALF), jnp.float32)

    t1 = _propagate(t0, ip3, zrows)
    t2 = _propagate(t1, ip3, zrows)
    t3 = _propagate(t2, ip3, zrows)
    u = _users_gather(t0, t1, t2, t3, users.astype(jnp.int32))

    return _rating(t0.reshape(2, ROWS, HALF), t1.reshape(2, ROWS, HALF),
                   t2.reshape(2, ROWS, HALF), t3.reshape(2, ROWS, HALF),
                   u.reshape(2, NBATCH, HALF))
